# Initial kernel scaffold; baseline (speedup 1.0000x reference)
import functools

import jax
import jax.numpy as jnp
from jax import lax
from jax.experimental import pallas as pl
from jax.experimental.pallas import tpu as pltpu
from jax.experimental.pallas import tpu_sc as plsc

N = 10000
D = 128
E = 320000
NC = 2
NS = 16
NW = NC * NS
CB = 128
CPW = 80
EP = NW * CB * CPW
PAD = EP - E
N2 = N + 16
RZ = N2 // NS
RPT = N // NS
DEGW = 16

_MESH = plsc.VectorSubcoreMesh(core_axis_name="c", subcore_axis_name="s")


def _make_agg(with_deg):
    out_type = [jax.ShapeDtypeStruct((NC, N, D), jnp.float32)]
    scratch = [
        pltpu.VMEM((CB,), jnp.int32),
        pltpu.VMEM((CB,), jnp.int32),
        pltpu.VMEM((CB, D), jnp.float32),
        pltpu.VMEM_SHARED((N2, D), jnp.float32),
        pltpu.SemaphoreType.DMA,
    ]
    if with_deg:
        out_type.append(jax.ShapeDtypeStruct((NC, N, DEGW), jnp.float32))
        scratch += [
            pltpu.VMEM((CB, DEGW), jnp.float32),
            pltpu.VMEM_SHARED((N2, DEGW), jnp.float32)
        ]

    def body(x_hbm, src_hbm, dst_hbm, zrow_hbm, zdeg_hbm, ones_hbm, *refs):
        if with_deg:
            (agg_out, deg_out, idx_s, idx_d, rows, agg_sh, sem,
             ones_v, deg_sh) = refs
        else:
            (agg_out, idx_s, idx_d, rows, agg_sh, sem) = refs
        c = lax.axis_index("c")
        s = lax.axis_index("s")
        w = c * NS + s

        pltpu.sync_copy(zrow_hbm, agg_sh.at[pl.ds(s * RZ, RZ)])
        if with_deg:
            pltpu.sync_copy(zdeg_hbm, deg_sh.at[pl.ds(s * RZ, RZ)])
            pltpu.sync_copy(ones_hbm, ones_v)
        plsc.subcore_barrier()

        def chunk(i, carry):
            base = (w * CPW + i) * CB
            pltpu.sync_copy(src_hbm.at[pl.ds(base, CB)], idx_s)
            pltpu.sync_copy(dst_hbm.at[pl.ds(base, CB)], idx_d)
            pltpu.async_copy(x_hbm.at[idx_s], rows, sem).wait()
            pltpu.sync_copy(rows, agg_sh.at[idx_d], add=True)
            if with_deg:
                pltpu.sync_copy(ones_v, deg_sh.at[idx_d], add=True)
            return carry

        lax.fori_loop(0, CPW, chunk, 0)
        plsc.subcore_barrier()
        pltpu.sync_copy(agg_sh.at[pl.ds(s * RPT, RPT)],
                        agg_out.at[c, pl.ds(s * RPT, RPT)])
        if with_deg:
            pltpu.sync_copy(deg_sh.at[pl.ds(s * RPT, RPT)],
                            deg_out.at[c, pl.ds(s * RPT, RPT)])

    return pl.kernel(body, out_type=out_type, mesh=_MESH,
                     scratch_types=scratch)


_agg_deg = _make_agg(True)
_agg_only = _make_agg(False)


def _scores_kernel():
    out_type = [jax.ShapeDtypeStruct((EP,), jnp.float32),
                jax.ShapeDtypeStruct((EP,), jnp.float32)]
    scratch = [
        pltpu.VMEM((CB,), jnp.int32),
        pltpu.VMEM((CB,), jnp.int32),
        pltpu.VMEM((CB, D), jnp.float32),
        pltpu.VMEM((CB, D), jnp.float32),
        pltpu.VMEM((CB,), jnp.float32),
        pltpu.SemaphoreType.DMA,
    ]

    def body(h_hbm, src_hbm, dst_hbm, nsrc_hbm, ndst_hbm, pos_out, neg_out,
             idx_a, idx_b, rows_a, rows_b, out_v, sem):
        c = lax.axis_index("c")
        s = lax.axis_index("s")
        w = c * NS + s
        lanes = lax.broadcasted_iota(jnp.int32, (16,), 0)

        def edge_set(sref, dref, oref):
            def chunk(i, carry):
                base = (w * CPW + i) * CB
                pltpu.sync_copy(sref.at[pl.ds(base, CB)], idx_a)
                pltpu.sync_copy(dref.at[pl.ds(base, CB)], idx_b)
                ca = pltpu.async_copy(h_hbm.at[idx_a], rows_a, sem)
                cb = pltpu.async_copy(h_hbm.at[idx_b], rows_b, sem)
                ca.wait()
                cb.wait()

                def group(g, carry2):
                    vec = jnp.zeros((16,), jnp.float32)
                    for j in range(16):
                        e = g * 16 + j
                        acc = (rows_a[e, pl.ds(0, 16)] *
                               rows_b[e, pl.ds(0, 16)])
                        for cc in range(1, D // 16):
                            acc = acc + (rows_a[e, pl.ds(cc * 16, 16)] *
                                         rows_b[e, pl.ds(cc * 16, 16)])
                        sc = jnp.sum(acc)
                        vec = jnp.where(lanes == j, sc, vec)
                    out_v[pl.ds(g * 16, 16)] = vec
                    return carry2

                lax.fori_loop(0, CB // 16, group, 0)
                pltpu.sync_copy(out_v, oref.at[pl.ds(base, CB)])
                return carry

            lax.fori_loop(0, CPW, chunk, 0)

        edge_set(src_hbm, dst_hbm, pos_out)
        edge_set(nsrc_hbm, ndst_hbm, neg_out)

    return pl.kernel(body, out_type=out_type, mesh=_MESH,
                     scratch_types=scratch)


_scores = _scores_kernel()


def _tc_layer(aggp, degp, x, ws_t, wn_t, b, relu):
    R = 500

    def body(agg_ref, deg_ref, x_ref, ws_ref, wn_ref, b_ref, o_ref):
        agg = agg_ref[0] + agg_ref[1]
        deg = deg_ref[0, :, 0:1] + deg_ref[1, :, 0:1]
        mean = agg / jnp.maximum(deg, 1.0)
        h = (jnp.dot(x_ref[...], ws_ref[...],
                     preferred_element_type=jnp.float32) +
             jnp.dot(mean, wn_ref[...], preferred_element_type=jnp.float32) +
             b_ref[...])
        o_ref[...] = jnp.maximum(h, 0.0) if relu else h

    return pl.pallas_call(
        body,
        grid=(N // R,),
        in_specs=[
            pl.BlockSpec((NC, R, D), lambda i: (0, i, 0)),
            pl.BlockSpec((NC, R, DEGW), lambda i: (0, i, 0)),
            pl.BlockSpec((R, D), lambda i: (i, 0)),
            pl.BlockSpec((D, D), lambda i: (0, 0)),
            pl.BlockSpec((D, D), lambda i: (0, 0)),
            pl.BlockSpec((1, D), lambda i: (0, 0)),
        ],
        out_specs=pl.BlockSpec((R, D), lambda i: (i, 0)),
        out_shape=jax.ShapeDtypeStruct((N, D), jnp.float32),
    )(aggp, degp, x, ws_t, wn_t, b)


def kernel(feat, edge_index, neg_edge_index, W1_self, b1_self, W1_neigh,
           W2_self, b2_self, W2_neigh):
    src = edge_index[0].astype(jnp.int32)
    dst = edge_index[1].astype(jnp.int32)
    nsrc = neg_edge_index[0].astype(jnp.int32)
    ndst = neg_edge_index[1].astype(jnp.int32)

    ar = jnp.arange(PAD, dtype=jnp.int32)
    pad_gather = (ar * 131) % N
    pad_trash = N + (ar % 16)
    srcp = jnp.concatenate([src, pad_gather])
    dstp = jnp.concatenate([dst, pad_trash])
    s_src = jnp.concatenate([src, pad_gather])
    s_dst = jnp.concatenate([dst, pad_gather])
    s_nsrc = jnp.concatenate([nsrc, pad_gather])
    s_ndst = jnp.concatenate([ndst, pad_gather])

    zrow = jnp.zeros((RZ, D), jnp.float32)
    zdeg = jnp.zeros((RZ, DEGW), jnp.float32)
    ones = jnp.ones((CB, DEGW), jnp.float32)

    agg1p, degp = _agg_deg(feat, srcp, dstp, zrow, zdeg, ones)
    h1 = _tc_layer(agg1p, degp, feat, W1_self.T, W1_neigh.T,
                   b1_self.reshape(1, D), True)
    agg2p = _agg_only(h1, srcp, dstp, zrow, zdeg, ones)
    h2 = _tc_layer(agg2p, degp, h1, W2_self.T, W2_neigh.T,
                   b2_self.reshape(1, D), False)
    pos, neg = _scores(h2, s_src, s_dst, s_nsrc, s_ndst)
    return pos[:E, None], neg[:E, None]


# speedup vs baseline: 1.9449x; 1.9449x over previous
import functools

import jax
import jax.numpy as jnp
from jax import lax
from jax.experimental import pallas as pl
from jax.experimental.pallas import tpu as pltpu
from jax.experimental.pallas import tpu_sc as plsc

N = 10000
D = 128
E = 320000
NC = 2
NS = 16
NW = NC * NS
CB = 128
CPW = 80
EP = NW * CB * CPW
PAD = EP - E
EPH = EP // NC
NCH = EPH // CB
N2 = 10112
RPT = N2 // NS
AR = RPT + 8
LCAP = 11264
LCH = LCAP // CB
BIGDST = 1 << 28


@functools.lru_cache(maxsize=None)
def _mesh():
    return plsc.VectorSubcoreMesh(core_axis_name="c", subcore_axis_name="s",
                                  num_cores=NC, num_subcores=NS)


def _sc_kernel(body, out_type, scratch):
    return pl.kernel(body, out_type=out_type, mesh=_mesh(),
                     scratch_types=scratch,
                     compiler_params=pltpu.CompilerParams(
                         needs_layout_passes=False))


@functools.lru_cache(maxsize=None)
def _make_route():
    out_type = [jax.ShapeDtypeStruct((NW * LCAP,), jnp.int32),
                jax.ShapeDtypeStruct((NW * LCAP,), jnp.int32),
                jax.ShapeDtypeStruct((NW * AR, 16), jnp.float32)]
    scratch = [
        pltpu.VMEM((CB,), jnp.int32),
        pltpu.VMEM((CB,), jnp.int32),
        pltpu.VMEM((LCAP + 16,), jnp.int32),
        pltpu.VMEM((LCAP + 16,), jnp.int32),
        pltpu.VMEM((AR, 16), jnp.float32),
        pltpu.SemaphoreType.DMA,
    ]

    def body(src_hbm, dst_hbm, zdeg_hbm, lsrc_out, ldst_out, deg_out,
             src_v, dst_v, lsrc, ldst, dega, sem):
        c = lax.axis_index("c")
        s = lax.axis_index("s")
        w = c * NS + s
        lanes = lax.broadcasted_iota(jnp.int32, (16,), 0)
        lo = s * RPT
        hi = lo + RPT

        for q in range(AR // CB):
            pltpu.sync_copy(zdeg_hbm, dega.at[pl.ds(q * CB, CB)])

        dummy_dst = RPT + (lanes & 7)

        def prefill(k, carry):
            dummy_src = ((k * 16 + lanes) * 131) & 8191
            lsrc[pl.ds(k * 16, 16)] = dummy_src
            ldst[pl.ds(k * 16, 16)] = dummy_dst
            return carry

        lax.fori_loop(0, LCAP // 16, prefill, 0)

        def chunk(i, cnt):
            base = c * EPH + i * CB
            pltpu.sync_copy(src_hbm.at[pl.ds(base, CB)], src_v)
            pltpu.sync_copy(dst_hbm.at[pl.ds(base, CB)], dst_v)
            for g in range(CB // 16):
                s16 = src_v[pl.ds(g * 16, 16)]
                d16 = dst_v[pl.ds(g * 16, 16)]
                inr = (d16 >= lo) & (d16 < hi)
                dloc = jnp.where(inr, d16 - lo, 0)
                plsc.addupdate_scatter(dega, [dloc, lanes],
                                       jnp.ones((16,), jnp.float32),
                                       mask=inr)
                plsc.store_compressed(lsrc.at[pl.ds(cnt, 16)], s16,
                                      mask=inr)
                plsc.store_compressed(ldst.at[pl.ds(cnt, 16)], dloc,
                                      mask=inr)
                cnt = cnt + jnp.max(plsc.all_reduce_population_count(inr))
            return cnt

        lax.fori_loop(0, NCH, chunk, jnp.int32(0))

        pltpu.sync_copy(lsrc.at[pl.ds(0, LCAP)],
                        lsrc_out.at[pl.ds(w * LCAP, LCAP)])
        pltpu.sync_copy(ldst.at[pl.ds(0, LCAP)],
                        ldst_out.at[pl.ds(w * LCAP, LCAP)])
        pltpu.sync_copy(dega, deg_out.at[pl.ds(w * AR, AR)])

    return _sc_kernel(body, out_type, scratch)


@functools.lru_cache(maxsize=None)
def _make_agg():
    out_type = [jax.ShapeDtypeStruct((NW * AR, D), jnp.float32)]
    scratch = [
        pltpu.VMEM((CB,), jnp.int32),
        pltpu.VMEM((CB,), jnp.int32),
        pltpu.VMEM((CB, D), jnp.float32),
        pltpu.VMEM((AR, D), jnp.float32),
        pltpu.SemaphoreType.DMA,
    ]

    def body(x_hbm, lsrc_hbm, ldst_hbm, zrow_hbm,
             agg_out, idx_v, dl_v, rows, acc, sem):
        c = lax.axis_index("c")
        s = lax.axis_index("s")
        w = c * NS + s
        lanes = lax.broadcasted_iota(jnp.int32, (16,), 0)

        for q in range(AR // CB):
            pltpu.sync_copy(zrow_hbm, acc.at[pl.ds(q * CB, CB)])

        def chunk(i, carry):
            base = w * LCAP + i * CB
            pltpu.sync_copy(lsrc_hbm.at[pl.ds(base, CB)], idx_v)
            pltpu.sync_copy(ldst_hbm.at[pl.ds(base, CB)], dl_v)
            pltpu.async_copy(x_hbm.at[idx_v], rows, sem).wait()

            def group(g, carry2):
                e16 = g * 16 + lanes
                dl16 = dl_v[pl.ds(g * 16, 16)]
                for t in range(16):
                    csh = (lanes + t) & 15
                    for cc in range(D // 16):
                        col = csh + cc * 16
                        vals = plsc.load_gather(rows, [e16, col])
                        plsc.addupdate_scatter(acc, [dl16, col], vals)
                return carry2

            lax.fori_loop(0, CB // 16, group, 0)
            return carry

        lax.fori_loop(0, LCH, chunk, 0)
        pltpu.sync_copy(acc, agg_out.at[pl.ds(w * AR, AR)])

    return _sc_kernel(body, out_type, scratch)


@functools.lru_cache(maxsize=None)
def _make_scores():
    out_type = [jax.ShapeDtypeStruct((EP,), jnp.float32),
                jax.ShapeDtypeStruct((EP,), jnp.float32)]
    scratch = [
        pltpu.VMEM((CB,), jnp.int32),
        pltpu.VMEM((CB,), jnp.int32),
        pltpu.VMEM((CB, D), jnp.float32),
        pltpu.VMEM((CB, D), jnp.float32),
        pltpu.VMEM((CB,), jnp.float32),
        pltpu.SemaphoreType.DMA,
    ]

    def body(h_hbm, src_hbm, dst_hbm, nsrc_hbm, ndst_hbm, pos_out, neg_out,
             idx_a, idx_b, rows_a, rows_b, out_v, sem):
        c = lax.axis_index("c")
        s = lax.axis_index("s")
        w = c * NS + s
        lanes = lax.broadcasted_iota(jnp.int32, (16,), 0)

        def edge_set(sref, dref, oref):
            def chunk(i, carry):
                base = (w * CPW + i) * CB
                pltpu.sync_copy(sref.at[pl.ds(base, CB)], idx_a)
                pltpu.sync_copy(dref.at[pl.ds(base, CB)], idx_b)
                ca = pltpu.async_copy(h_hbm.at[idx_a], rows_a, sem)
                cb = pltpu.async_copy(h_hbm.at[idx_b], rows_b, sem)
                ca.wait()
                cb.wait()

                def group(g, carry2):
                    edge_ids = g * 16 + lanes
                    score = jnp.zeros((16,), jnp.float32)
                    for t in range(D):
                        col = (lanes + t) & (D - 1)
                        a = plsc.load_gather(rows_a, [edge_ids, col])
                        b = plsc.load_gather(rows_b, [edge_ids, col])
                        score = score + a * b
                    out_v[pl.ds(g * 16, 16)] = score
                    return carry2

                lax.fori_loop(0, CB // 16, group, 0)
                pltpu.sync_copy(out_v, oref.at[pl.ds(base, CB)])
                return carry

            lax.fori_loop(0, CPW, chunk, 0)

        edge_set(src_hbm, dst_hbm, pos_out)
        edge_set(nsrc_hbm, ndst_hbm, neg_out)

    return _sc_kernel(body, out_type, scratch)


def _tc_layer(aggp, degp, x, ws_t, wn_t, b, relu):
    R = 1000

    def body(agg_ref, deg_ref, x_ref, ws_ref, wn_ref, b_ref, o_ref):
        agg = agg_ref[0] + agg_ref[1]
        deg = jnp.sum(deg_ref[0] + deg_ref[1], axis=-1, keepdims=True)
        mean = agg / jnp.maximum(deg, 1.0)
        h = (jnp.dot(x_ref[...], ws_ref[...],
                     preferred_element_type=jnp.float32) +
             jnp.dot(mean, wn_ref[...], preferred_element_type=jnp.float32) +
             b_ref[...])
        o_ref[...] = jnp.maximum(h, 0.0) if relu else h

    return pl.pallas_call(
        body,
        grid=(N // R,),
        in_specs=[
            pl.BlockSpec((NC, R, D), lambda i: (0, i, 0)),
            pl.BlockSpec((NC, R, 16), lambda i: (0, i, 0)),
            pl.BlockSpec((R, D), lambda i: (i, 0)),
            pl.BlockSpec((D, D), lambda i: (0, 0)),
            pl.BlockSpec((D, D), lambda i: (0, 0)),
            pl.BlockSpec((1, D), lambda i: (0, 0)),
        ],
        out_specs=pl.BlockSpec((R, D), lambda i: (i, 0)),
        out_shape=jax.ShapeDtypeStruct((N, D), jnp.float32),
    )(aggp, degp, x, ws_t, wn_t, b)


def _trim(part, width):
    return part.reshape(NC, NS, AR, width)[:, :, :RPT, :].reshape(
        NC, N2, width)


def kernel(feat, edge_index, neg_edge_index, W1_self, b1_self, W1_neigh,
           W2_self, b2_self, W2_neigh):
    src = edge_index[0].astype(jnp.int32)
    dst = edge_index[1].astype(jnp.int32)
    nsrc = neg_edge_index[0].astype(jnp.int32)
    ndst = neg_edge_index[1].astype(jnp.int32)

    ar = jnp.arange(PAD, dtype=jnp.int32)
    pad_gather = (ar * 131) % N
    pad_drop = jnp.full((PAD,), BIGDST, jnp.int32)
    srcp = jnp.concatenate([src, pad_gather])
    dstp = jnp.concatenate([dst, pad_drop])
    s_src = jnp.concatenate([src, pad_gather])
    s_dst = jnp.concatenate([dst, pad_gather])
    s_nsrc = jnp.concatenate([nsrc, pad_gather])
    s_ndst = jnp.concatenate([ndst, pad_gather])

    zrow = jnp.zeros((CB, D), jnp.float32)
    zdeg = jnp.zeros((CB, 16), jnp.float32)

    lsrc, ldst, degp = _make_route()(srcp, dstp, zdeg)
    degp = _trim(degp, 16)
    agg1p, = _make_agg()(feat, lsrc, ldst, zrow)
    agg1p = _trim(agg1p, D)
    h1 = _tc_layer(agg1p, degp, feat, W1_self.T, W1_neigh.T,
                   b1_self.reshape(1, D), True)
    agg2p, = _make_agg()(h1, lsrc, ldst, zrow)
    agg2p = _trim(agg2p, D)
    h2 = _tc_layer(agg2p, degp, h1, W2_self.T, W2_neigh.T,
                   b2_self.reshape(1, D), False)
    pos, neg = _make_scores()(h2, s_src, s_dst, s_nsrc, s_ndst)
    return pos[:E, None], neg[:E, None]


# speedup vs baseline: 2.7513x; 1.4146x over previous
import functools

import jax
import jax.numpy as jnp
from jax import lax
from jax.experimental import pallas as pl
from jax.experimental.pallas import tpu as pltpu
from jax.experimental.pallas import tpu_sc as plsc

N = 10000
D = 128
E = 320000
NC = 2
NS = 16
NW = NC * NS
CB = 128
CPW = 80
EP = NW * CB * CPW
PAD = EP - E
EPH = EP // NC
NCH = EPH // CB
N2 = 10112
RPT = N2 // NS
AR = RPT + 8
LCAP = 11264
LCH = LCAP // CB
BIGDST = 1 << 28
RBK = 2048
GRP = 128


@functools.lru_cache(maxsize=None)
def _mesh():
    return plsc.VectorSubcoreMesh(core_axis_name="c", subcore_axis_name="s",
                                  num_cores=NC, num_subcores=NS)


def _sc_kernel(body, out_type, scratch):
    return pl.kernel(body, out_type=out_type, mesh=_mesh(),
                     scratch_types=scratch,
                     compiler_params=pltpu.CompilerParams(
                         needs_layout_passes=False))


@functools.lru_cache(maxsize=None)
def _make_route():
    out_type = [jax.ShapeDtypeStruct((NW * LCAP,), jnp.int32),
                jax.ShapeDtypeStruct((NW * LCAP,), jnp.int32),
                jax.ShapeDtypeStruct((NW * AR, 16), jnp.float32)]
    scratch = [
        pltpu.VMEM((RBK,), jnp.int32),
        pltpu.VMEM((RBK,), jnp.int32),
        pltpu.VMEM((LCAP + 16,), jnp.int32),
        pltpu.VMEM((LCAP + 16,), jnp.int32),
        pltpu.VMEM((AR, 16), jnp.float32),
        pltpu.SemaphoreType.DMA,
    ]

    def body(src_hbm, dst_hbm, zdeg_hbm, lsrc_out, ldst_out, deg_out,
             src_v, dst_v, lsrc, ldst, dega, sem):
        c = lax.axis_index("c")
        s = lax.axis_index("s")
        w = c * NS + s
        lanes = lax.broadcasted_iota(jnp.int32, (16,), 0)
        lo = s * RPT
        hi = lo + RPT

        for q in range(AR // CB):
            pltpu.sync_copy(zdeg_hbm, dega.at[pl.ds(q * CB, CB)])

        dummy_dst = RPT + (lanes & 7)

        def prefill(k, carry):
            dummy_src = ((k * 16 + lanes) * 131) & 8191
            lsrc[pl.ds(k * 16, 16)] = dummy_src
            ldst[pl.ds(k * 16, 16)] = dummy_dst
            return carry

        lax.fori_loop(0, LCAP // 16, prefill, 0)

        def chunk(i, cnt):
            base = c * EPH + i * RBK
            pltpu.sync_copy(src_hbm.at[pl.ds(base, RBK)], src_v)
            pltpu.sync_copy(dst_hbm.at[pl.ds(base, RBK)], dst_v)

            def sub(j, cnt2):
                return process(j, cnt2)
            return lax.fori_loop(0, RBK // GRP, sub, cnt)

        def process(g, cnt):
            if True:
                for u in range(GRP // 16):
                    s16 = src_v[pl.ds(g * GRP + u * 16, 16)]
                    d16 = dst_v[pl.ds(g * GRP + u * 16, 16)]
                    inr = (d16 >= lo) & (d16 < hi)
                    dloc = jnp.where(inr, d16 - lo, 0)
                    plsc.addupdate_scatter(dega, [dloc, lanes],
                                           jnp.ones((16,), jnp.float32),
                                           mask=inr)
                    plsc.store_compressed(lsrc.at[pl.ds(cnt, 16)], s16,
                                          mask=inr)
                    plsc.store_compressed(ldst.at[pl.ds(cnt, 16)], dloc,
                                          mask=inr)
                    cnt = cnt + jnp.max(
                        plsc.all_reduce_population_count(inr))
            return cnt

        lax.fori_loop(0, EPH // RBK, chunk, jnp.int32(0))

        pltpu.sync_copy(lsrc.at[pl.ds(0, LCAP)],
                        lsrc_out.at[pl.ds(w * LCAP, LCAP)])
        pltpu.sync_copy(ldst.at[pl.ds(0, LCAP)],
                        ldst_out.at[pl.ds(w * LCAP, LCAP)])
        pltpu.sync_copy(dega, deg_out.at[pl.ds(w * AR, AR)])

    return _sc_kernel(body, out_type, scratch)


@functools.lru_cache(maxsize=None)
def _make_agg():
    out_type = [jax.ShapeDtypeStruct((NW * AR, D), jnp.float32)]
    scratch = [
        pltpu.VMEM((CB,), jnp.int32),
        pltpu.VMEM((CB,), jnp.int32),
        pltpu.VMEM((CB, D), jnp.float32),
        pltpu.VMEM((AR, D), jnp.float32),
        pltpu.SemaphoreType.DMA,
    ]

    def body(x_hbm, lsrc_hbm, ldst_hbm, zrow_hbm,
             agg_out, idx_v, dl_v, rows, acc, sem):
        c = lax.axis_index("c")
        s = lax.axis_index("s")
        w = c * NS + s
        lanes = lax.broadcasted_iota(jnp.int32, (16,), 0)

        for q in range(AR // CB):
            pltpu.sync_copy(zrow_hbm, acc.at[pl.ds(q * CB, CB)])

        def chunk(i, carry):
            base = w * LCAP + i * CB
            pltpu.sync_copy(lsrc_hbm.at[pl.ds(base, CB)], idx_v)
            pltpu.sync_copy(ldst_hbm.at[pl.ds(base, CB)], dl_v)
            pltpu.async_copy(x_hbm.at[idx_v], rows, sem).wait()

            def group(g, carry2):
                e16 = g * 16 + lanes
                dl16 = dl_v[pl.ds(g * 16, 16)]
                for t in range(16):
                    csh = (lanes + t) & 15
                    for cc in range(D // 16):
                        col = csh + cc * 16
                        vals = plsc.load_gather(rows, [e16, col])
                        plsc.addupdate_scatter(acc, [dl16, col], vals)
                return carry2

            lax.fori_loop(0, CB // 16, group, 0)
            return carry

        lax.fori_loop(0, LCH, chunk, 0)
        pltpu.sync_copy(acc, agg_out.at[pl.ds(w * AR, AR)])

    return _sc_kernel(body, out_type, scratch)


@functools.lru_cache(maxsize=None)
def _make_scores():
    out_type = [jax.ShapeDtypeStruct((EP,), jnp.float32),
                jax.ShapeDtypeStruct((EP,), jnp.float32)]
    scratch = [
        pltpu.VMEM((CB,), jnp.int32),
        pltpu.VMEM((CB,), jnp.int32),
        pltpu.VMEM((CB, D), jnp.float32),
        pltpu.VMEM((CB, D), jnp.float32),
        pltpu.VMEM((CB,), jnp.float32),
        pltpu.SemaphoreType.DMA,
    ]

    def body(h_hbm, src_hbm, dst_hbm, nsrc_hbm, ndst_hbm, pos_out, neg_out,
             idx_a, idx_b, rows_a, rows_b, out_v, sem):
        c = lax.axis_index("c")
        s = lax.axis_index("s")
        w = c * NS + s
        lanes = lax.broadcasted_iota(jnp.int32, (16,), 0)

        def edge_set(sref, dref, oref):
            def chunk(i, carry):
                base = (w * CPW + i) * CB
                pltpu.sync_copy(sref.at[pl.ds(base, CB)], idx_a)
                pltpu.sync_copy(dref.at[pl.ds(base, CB)], idx_b)
                ca = pltpu.async_copy(h_hbm.at[idx_a], rows_a, sem)
                cb = pltpu.async_copy(h_hbm.at[idx_b], rows_b, sem)
                ca.wait()
                cb.wait()

                def group(g, carry2):
                    edge_ids = g * 16 + lanes
                    score = jnp.zeros((16,), jnp.float32)
                    for t in range(D):
                        col = (lanes + t) & (D - 1)
                        a = plsc.load_gather(rows_a, [edge_ids, col])
                        b = plsc.load_gather(rows_b, [edge_ids, col])
                        score = score + a * b
                    out_v[pl.ds(g * 16, 16)] = score
                    return carry2

                lax.fori_loop(0, CB // 16, group, 0)
                pltpu.sync_copy(out_v, oref.at[pl.ds(base, CB)])
                return carry

            lax.fori_loop(0, CPW, chunk, 0)

        edge_set(src_hbm, dst_hbm, pos_out)
        edge_set(nsrc_hbm, ndst_hbm, neg_out)

    return _sc_kernel(body, out_type, scratch)


def _tc_layer(aggp, degp, x, ws_t, wn_t, b, relu):
    R = 1000

    def body(agg_ref, deg_ref, x_ref, ws_ref, wn_ref, b_ref, o_ref):
        agg = agg_ref[0] + agg_ref[1]
        deg = jnp.sum(deg_ref[0] + deg_ref[1], axis=-1, keepdims=True)
        mean = agg / jnp.maximum(deg, 1.0)
        h = (jnp.dot(x_ref[...], ws_ref[...],
                     preferred_element_type=jnp.float32) +
             jnp.dot(mean, wn_ref[...], preferred_element_type=jnp.float32) +
             b_ref[...])
        o_ref[...] = jnp.maximum(h, 0.0) if relu else h

    return pl.pallas_call(
        body,
        grid=(N // R,),
        in_specs=[
            pl.BlockSpec((NC, R, D), lambda i: (0, i, 0)),
            pl.BlockSpec((NC, R, 16), lambda i: (0, i, 0)),
            pl.BlockSpec((R, D), lambda i: (i, 0)),
            pl.BlockSpec((D, D), lambda i: (0, 0)),
            pl.BlockSpec((D, D), lambda i: (0, 0)),
            pl.BlockSpec((1, D), lambda i: (0, 0)),
        ],
        out_specs=pl.BlockSpec((R, D), lambda i: (i, 0)),
        out_shape=jax.ShapeDtypeStruct((N, D), jnp.float32),
    )(aggp, degp, x, ws_t, wn_t, b)


def _trim(part, width):
    return part.reshape(NC, NS, AR, width)[:, :, :RPT, :].reshape(
        NC, N2, width)


def kernel(feat, edge_index, neg_edge_index, W1_self, b1_self, W1_neigh,
           W2_self, b2_self, W2_neigh):
    src = edge_index[0].astype(jnp.int32)
    dst = edge_index[1].astype(jnp.int32)
    nsrc = neg_edge_index[0].astype(jnp.int32)
    ndst = neg_edge_index[1].astype(jnp.int32)

    ar = jnp.arange(PAD, dtype=jnp.int32)
    pad_gather = (ar * 131) % N
    pad_drop = jnp.full((PAD,), BIGDST, jnp.int32)
    srcp = jnp.concatenate([src, pad_gather])
    dstp = jnp.concatenate([dst, pad_drop])
    s_src = jnp.concatenate([src, pad_gather])
    s_dst = jnp.concatenate([dst, pad_gather])
    s_nsrc = jnp.concatenate([nsrc, pad_gather])
    s_ndst = jnp.concatenate([ndst, pad_gather])

    zrow = jnp.zeros((CB, D), jnp.float32)
    zdeg = jnp.zeros((CB, 16), jnp.float32)

    lsrc, ldst, degp = _make_route()(srcp, dstp, zdeg)
    degp = _trim(degp, 16)
    agg1p, = _make_agg()(feat, lsrc, ldst, zrow)
    agg1p = _trim(agg1p, D)
    h1 = _tc_layer(agg1p, degp, feat, W1_self.T, W1_neigh.T,
                   b1_self.reshape(1, D), True)
    agg2p, = _make_agg()(h1, lsrc, ldst, zrow)
    agg2p = _trim(agg2p, D)
    h2 = _tc_layer(agg2p, degp, h1, W2_self.T, W2_neigh.T,
                   b2_self.reshape(1, D), False)
    pos, neg = _make_scores()(h2, s_src, s_dst, s_nsrc, s_ndst)
    return pos[:E, None], neg[:E, None]


# speedup vs baseline: 3.0701x; 1.1159x over previous
import functools

import jax
import jax.numpy as jnp
from jax import lax
from jax.experimental import pallas as pl
from jax.experimental.pallas import tpu as pltpu
from jax.experimental.pallas import tpu_sc as plsc

N = 10000
D = 128
E = 320000
NC = 2
NS = 16
NW = NC * NS
CB = 128
CPW = 80
EP = NW * CB * CPW
PAD = EP - E
EPH = EP // NC
NCH = EPH // CB
N2 = 10112
RPT = N2 // NS
AR = RPT + 8
LCAP = 11264
LCH = LCAP // CB
BIGDST = 1 << 28
RBK = 2048
GRP = 128


@functools.lru_cache(maxsize=None)
def _mesh():
    return plsc.VectorSubcoreMesh(core_axis_name="c", subcore_axis_name="s",
                                  num_cores=NC, num_subcores=NS)


def _sc_kernel(body, out_type, scratch):
    return pl.kernel(body, out_type=out_type, mesh=_mesh(),
                     scratch_types=scratch,
                     compiler_params=pltpu.CompilerParams(
                         needs_layout_passes=False))


@functools.lru_cache(maxsize=None)
def _make_route():
    out_type = [jax.ShapeDtypeStruct((NW * LCAP,), jnp.int32),
                jax.ShapeDtypeStruct((NW * LCAP,), jnp.int32),
                jax.ShapeDtypeStruct((NW * AR, 16), jnp.float32)]
    scratch = [
        pltpu.VMEM((RBK,), jnp.int32),
        pltpu.VMEM((RBK,), jnp.int32),
        pltpu.VMEM((LCAP + 16,), jnp.int32),
        pltpu.VMEM((LCAP + 16,), jnp.int32),
        pltpu.VMEM((AR, 16), jnp.float32),
        pltpu.SemaphoreType.DMA,
    ]

    def body(src_hbm, dst_hbm, zdeg_hbm, lsrc_out, ldst_out, deg_out,
             src_v, dst_v, lsrc, ldst, dega, sem):
        c = lax.axis_index("c")
        s = lax.axis_index("s")
        w = c * NS + s
        lanes = lax.broadcasted_iota(jnp.int32, (16,), 0)
        lo = s * RPT
        hi = lo + RPT

        for q in range(AR // CB):
            pltpu.sync_copy(zdeg_hbm, dega.at[pl.ds(q * CB, CB)])

        dummy_dst = RPT + (lanes & 7)

        def prefill(k, carry):
            dummy_src = ((k * 16 + lanes) * 131) & 8191
            lsrc[pl.ds(k * 16, 16)] = dummy_src
            ldst[pl.ds(k * 16, 16)] = dummy_dst
            return carry

        lax.fori_loop(0, LCAP // 16, prefill, 0)

        def chunk(i, cnt):
            base = c * EPH + i * RBK
            pltpu.sync_copy(src_hbm.at[pl.ds(base, RBK)], src_v)
            pltpu.sync_copy(dst_hbm.at[pl.ds(base, RBK)], dst_v)

            def sub(j, cnt2):
                return process(j, cnt2)
            return lax.fori_loop(0, RBK // GRP, sub, cnt)

        def process(g, cnt):
            if True:
                for u in range(GRP // 16):
                    s16 = src_v[pl.ds(g * GRP + u * 16, 16)]
                    d16 = dst_v[pl.ds(g * GRP + u * 16, 16)]
                    inr = (d16 >= lo) & (d16 < hi)
                    dloc = jnp.where(inr, d16 - lo, 0)
                    plsc.addupdate_scatter(dega, [dloc, lanes],
                                           jnp.ones((16,), jnp.float32),
                                           mask=inr)
                    plsc.store_compressed(lsrc.at[pl.ds(cnt, 16)], s16,
                                          mask=inr)
                    plsc.store_compressed(ldst.at[pl.ds(cnt, 16)], dloc,
                                          mask=inr)
                    cnt = cnt + jnp.max(
                        plsc.all_reduce_population_count(inr))
            return cnt

        lax.fori_loop(0, EPH // RBK, chunk, jnp.int32(0))

        pltpu.sync_copy(lsrc.at[pl.ds(0, LCAP)],
                        lsrc_out.at[pl.ds(w * LCAP, LCAP)])
        pltpu.sync_copy(ldst.at[pl.ds(0, LCAP)],
                        ldst_out.at[pl.ds(w * LCAP, LCAP)])
        pltpu.sync_copy(dega, deg_out.at[pl.ds(w * AR, AR)])

    return _sc_kernel(body, out_type, scratch)


@functools.lru_cache(maxsize=None)
def _make_agg():
    out_type = [jax.ShapeDtypeStruct((NW * AR, D), jnp.float32)]
    scratch = [
        pltpu.VMEM((CB,), jnp.int32),
        pltpu.VMEM((CB,), jnp.int32),
        pltpu.VMEM((CB, D), jnp.float32),
        pltpu.VMEM((AR, D), jnp.float32),
        pltpu.SemaphoreType.DMA,
    ]

    def body(x_hbm, lsrc_hbm, ldst_hbm, zrow_hbm,
             agg_out, idx_v, dl_v, rows, acc, sem):
        c = lax.axis_index("c")
        s = lax.axis_index("s")
        w = c * NS + s
        lanes = lax.broadcasted_iota(jnp.int32, (16,), 0)

        for q in range(AR // CB):
            pltpu.sync_copy(zrow_hbm, acc.at[pl.ds(q * CB, CB)])

        def chunk(i, carry):
            base = w * LCAP + i * CB
            pltpu.sync_copy(lsrc_hbm.at[pl.ds(base, CB)], idx_v)
            pltpu.sync_copy(ldst_hbm.at[pl.ds(base, CB)], dl_v)
            pltpu.async_copy(x_hbm.at[idx_v], rows, sem).wait()

            def group(g, carry2):
                e16 = g * 16 + lanes
                dl16 = dl_v[pl.ds(g * 16, 16)]
                for t in range(16):
                    csh = (lanes + t) & 15
                    for cc in range(D // 16):
                        col = csh + cc * 16
                        vals = plsc.load_gather(rows, [e16, col])
                        plsc.addupdate_scatter(acc, [dl16, col], vals)
                return carry2

            lax.fori_loop(0, CB // 16, group, 0)
            return carry

        lax.fori_loop(0, LCH, chunk, 0)
        pltpu.sync_copy(acc, agg_out.at[pl.ds(w * AR, AR)])

    return _sc_kernel(body, out_type, scratch)


@functools.lru_cache(maxsize=None)
def _make_scores():
    EPW = CPW * CB
    out_type = [jax.ShapeDtypeStruct((EP,), jnp.float32),
                jax.ShapeDtypeStruct((EP,), jnp.float32)]
    scratch = [
        pltpu.VMEM((EPW,), jnp.int32),
        pltpu.VMEM((EPW,), jnp.int32),
        pltpu.VMEM((CB, D), jnp.float32),
        pltpu.VMEM((CB, D), jnp.float32),
        pltpu.VMEM((CB, D), jnp.float32),
        pltpu.VMEM((CB, D), jnp.float32),
        pltpu.VMEM((EPW,), jnp.float32),
        pltpu.SemaphoreType.DMA,
        pltpu.SemaphoreType.DMA,
    ]

    def body(h_hbm, src_hbm, dst_hbm, nsrc_hbm, ndst_hbm, pos_out, neg_out,
             idx_a, idx_b, ra0, rb0, ra1, rb1, out_v, sem0, sem1):
        c = lax.axis_index("c")
        s = lax.axis_index("s")
        w = c * NS + s
        lanes = lax.broadcasted_iota(jnp.int32, (16,), 0)
        bufs = ((ra0, rb0, sem0), (ra1, rb1, sem1))

        def edge_set(sref, dref, oref):
            pltpu.sync_copy(sref.at[pl.ds(w * EPW, EPW)], idx_a)
            pltpu.sync_copy(dref.at[pl.ds(w * EPW, EPW)], idx_b)

            def start(i, b):
                ra, rb, sm = bufs[b]
                pltpu.async_copy(h_hbm.at[idx_a.at[pl.ds(i * CB, CB)]],
                                 ra, sm)
                pltpu.async_copy(h_hbm.at[idx_b.at[pl.ds(i * CB, CB)]],
                                 rb, sm)

            def wait(b):
                ra, rb, sm = bufs[b]
                pltpu.make_async_copy(
                    h_hbm.at[idx_a.at[pl.ds(0, CB)]], ra, sm).wait()
                pltpu.make_async_copy(
                    h_hbm.at[idx_b.at[pl.ds(0, CB)]], rb, sm).wait()

            def compute(i, b):
                ra, rb, _ = bufs[b]

                def group(g, carry2):
                    edge_ids = g * 16 + lanes
                    score = jnp.zeros((16,), jnp.float32)
                    for t in range(D):
                        col = (lanes + t) & (D - 1)
                        a = plsc.load_gather(ra, [edge_ids, col])
                        b2 = plsc.load_gather(rb, [edge_ids, col])
                        score = score + a * b2
                    out_v[pl.ds(i * CB + g * 16, 16)] = score
                    return carry2

                lax.fori_loop(0, CB // 16, group, 0)

            start(0, 0)

            def pair(p, carry):
                i0 = 2 * p
                start(i0 + 1, 1)
                wait(0)
                compute(i0, 0)

                @pl.when(p < CPW // 2 - 1)
                def _():
                    start(i0 + 2, 0)

                wait(1)
                compute(i0 + 1, 1)
                return carry

            lax.fori_loop(0, CPW // 2, pair, 0)
            pltpu.sync_copy(out_v, oref.at[pl.ds(w * EPW, EPW)])

        edge_set(src_hbm, dst_hbm, pos_out)
        edge_set(nsrc_hbm, ndst_hbm, neg_out)

    return _sc_kernel(body, out_type, scratch)


def _tc_layer(aggp, degp, x, ws_t, wn_t, b, relu):
    R = 1000

    def body(agg_ref, deg_ref, x_ref, ws_ref, wn_ref, b_ref, o_ref):
        agg = agg_ref[0] + agg_ref[1]
        deg = jnp.sum(deg_ref[0] + deg_ref[1], axis=-1, keepdims=True)
        mean = agg / jnp.maximum(deg, 1.0)
        h = (jnp.dot(x_ref[...], ws_ref[...],
                     preferred_element_type=jnp.float32) +
             jnp.dot(mean, wn_ref[...], preferred_element_type=jnp.float32) +
             b_ref[...])
        o_ref[...] = jnp.maximum(h, 0.0) if relu else h

    return pl.pallas_call(
        body,
        grid=(N // R,),
        in_specs=[
            pl.BlockSpec((NC, R, D), lambda i: (0, i, 0)),
            pl.BlockSpec((NC, R, 16), lambda i: (0, i, 0)),
            pl.BlockSpec((R, D), lambda i: (i, 0)),
            pl.BlockSpec((D, D), lambda i: (0, 0)),
            pl.BlockSpec((D, D), lambda i: (0, 0)),
            pl.BlockSpec((1, D), lambda i: (0, 0)),
        ],
        out_specs=pl.BlockSpec((R, D), lambda i: (i, 0)),
        out_shape=jax.ShapeDtypeStruct((N, D), jnp.float32),
    )(aggp, degp, x, ws_t, wn_t, b)


def _trim(part, width):
    return part.reshape(NC, NS, AR, width)[:, :, :RPT, :].reshape(
        NC, N2, width)


def kernel(feat, edge_index, neg_edge_index, W1_self, b1_self, W1_neigh,
           W2_self, b2_self, W2_neigh):
    src = edge_index[0].astype(jnp.int32)
    dst = edge_index[1].astype(jnp.int32)
    nsrc = neg_edge_index[0].astype(jnp.int32)
    ndst = neg_edge_index[1].astype(jnp.int32)

    ar = jnp.arange(PAD, dtype=jnp.int32)
    pad_gather = (ar * 131) % N
    pad_drop = jnp.full((PAD,), BIGDST, jnp.int32)
    srcp = jnp.concatenate([src, pad_gather])
    dstp = jnp.concatenate([dst, pad_drop])
    s_src = jnp.concatenate([src, pad_gather])
    s_dst = jnp.concatenate([dst, pad_gather])
    s_nsrc = jnp.concatenate([nsrc, pad_gather])
    s_ndst = jnp.concatenate([ndst, pad_gather])

    zrow = jnp.zeros((CB, D), jnp.float32)
    zdeg = jnp.zeros((CB, 16), jnp.float32)

    lsrc, ldst, degp = _make_route()(srcp, dstp, zdeg)
    degp = _trim(degp, 16)
    agg1p, = _make_agg()(feat, lsrc, ldst, zrow)
    agg1p = _trim(agg1p, D)
    h1 = _tc_layer(agg1p, degp, feat, W1_self.T, W1_neigh.T,
                   b1_self.reshape(1, D), True)
    agg2p, = _make_agg()(h1, lsrc, ldst, zrow)
    agg2p = _trim(agg2p, D)
    h2 = _tc_layer(agg2p, degp, h1, W2_self.T, W2_neigh.T,
                   b2_self.reshape(1, D), False)
    pos, neg = _make_scores()(h2, s_src, s_dst, s_nsrc, s_ndst)
    return pos[:E, None], neg[:E, None]


# speedup vs baseline: 3.8000x; 1.2377x over previous
import functools

import jax
import jax.numpy as jnp
from jax import lax
from jax.experimental import pallas as pl
from jax.experimental.pallas import tpu as pltpu
from jax.experimental.pallas import tpu_sc as plsc

N = 10000
D = 128
E = 320000
NC = 2
NS = 16
NW = NC * NS
CB = 128
CPW = 80
EP = NW * CB * CPW
PAD = EP - E
EPH = EP // NC
NCH = EPH // CB
N2 = 10112
RPT = N2 // NS
AR = RPT + 8
LCAP = 11264
LCH = LCAP // CB
BIGDST = 1 << 28
RBK = 2048
GRP = 128


@functools.lru_cache(maxsize=None)
def _mesh():
    return plsc.VectorSubcoreMesh(core_axis_name="c", subcore_axis_name="s",
                                  num_cores=NC, num_subcores=NS)


def _sc_kernel(body, out_type, scratch):
    return pl.kernel(body, out_type=out_type, mesh=_mesh(),
                     scratch_types=scratch,
                     compiler_params=pltpu.CompilerParams(
                         needs_layout_passes=False))


@functools.lru_cache(maxsize=None)
def _make_route():
    out_type = [jax.ShapeDtypeStruct((NW * LCAP,), jnp.int32),
                jax.ShapeDtypeStruct((NW * LCAP,), jnp.int32),
                jax.ShapeDtypeStruct((NW * AR, 16), jnp.float32)]
    scratch = [
        pltpu.VMEM((RBK,), jnp.int32),
        pltpu.VMEM((RBK,), jnp.int32),
        pltpu.VMEM((LCAP + 16,), jnp.int32),
        pltpu.VMEM((LCAP + 16,), jnp.int32),
        pltpu.VMEM((AR, 16), jnp.float32),
        pltpu.SemaphoreType.DMA,
    ]

    def body(src_hbm, dst_hbm, zdeg_hbm, lsrc_out, ldst_out, deg_out,
             src_v, dst_v, lsrc, ldst, dega, sem):
        c = lax.axis_index("c")
        s = lax.axis_index("s")
        w = c * NS + s
        lanes = lax.broadcasted_iota(jnp.int32, (16,), 0)
        lo = s * RPT
        hi = lo + RPT

        for q in range(AR // CB):
            pltpu.sync_copy(zdeg_hbm, dega.at[pl.ds(q * CB, CB)])

        dummy_dst = RPT + (lanes & 7)

        def prefill(k, carry):
            dummy_src = ((k * 16 + lanes) * 131) & 8191
            lsrc[pl.ds(k * 16, 16)] = dummy_src
            ldst[pl.ds(k * 16, 16)] = dummy_dst
            return carry

        lax.fori_loop(0, LCAP // 16, prefill, 0)

        def chunk(i, cnt):
            base = c * EPH + i * RBK
            pltpu.sync_copy(src_hbm.at[pl.ds(base, RBK)], src_v)
            pltpu.sync_copy(dst_hbm.at[pl.ds(base, RBK)], dst_v)

            def sub(j, cnt2):
                return process(j, cnt2)
            return lax.fori_loop(0, RBK // GRP, sub, cnt)

        def process(g, cnt):
            if True:
                for u in range(GRP // 16):
                    s16 = src_v[pl.ds(g * GRP + u * 16, 16)]
                    d16 = dst_v[pl.ds(g * GRP + u * 16, 16)]
                    inr = (d16 >= lo) & (d16 < hi)
                    dloc = jnp.where(inr, d16 - lo, 0)
                    plsc.addupdate_scatter(dega, [dloc, lanes],
                                           jnp.ones((16,), jnp.float32),
                                           mask=inr)
                    plsc.store_compressed(lsrc.at[pl.ds(cnt, 16)], s16,
                                          mask=inr)
                    plsc.store_compressed(ldst.at[pl.ds(cnt, 16)], dloc,
                                          mask=inr)
                    cnt = cnt + jnp.max(
                        plsc.all_reduce_population_count(inr))
            return cnt

        lax.fori_loop(0, EPH // RBK, chunk, jnp.int32(0))

        pltpu.sync_copy(lsrc.at[pl.ds(0, LCAP)],
                        lsrc_out.at[pl.ds(w * LCAP, LCAP)])
        pltpu.sync_copy(ldst.at[pl.ds(0, LCAP)],
                        ldst_out.at[pl.ds(w * LCAP, LCAP)])
        pltpu.sync_copy(dega, deg_out.at[pl.ds(w * AR, AR)])

    return _sc_kernel(body, out_type, scratch)


@functools.lru_cache(maxsize=None)
def _make_agg():
    LBK = 1024
    NBLK = LCAP // LBK
    out_type = [jax.ShapeDtypeStruct((NW * AR, D), jnp.float32)]
    scratch = [
        pltpu.VMEM((LBK,), jnp.int32),
        pltpu.VMEM((LBK,), jnp.int32),
        pltpu.VMEM((CB, D), jnp.float32),
        pltpu.VMEM((CB, D), jnp.float32),
        pltpu.VMEM((AR, D), jnp.float32),
        pltpu.SemaphoreType.DMA,
        pltpu.SemaphoreType.DMA,
    ]

    def body(x_hbm, lsrc_hbm, ldst_hbm, zrow_hbm,
             agg_out, idx_v, dl_v, rows0, rows1, acc, sem0, sem1):
        c = lax.axis_index("c")
        s = lax.axis_index("s")
        w = c * NS + s
        lanes = lax.broadcasted_iota(jnp.int32, (16,), 0)
        bufs = ((rows0, sem0), (rows1, sem1))

        for q in range(AR // CB):
            pltpu.sync_copy(zrow_hbm, acc.at[pl.ds(q * CB, CB)])

        def start(i, b):
            rows, sm = bufs[b]
            pltpu.async_copy(x_hbm.at[idx_v.at[pl.ds(i * CB, CB)]], rows, sm)

        def wait(b):
            rows, sm = bufs[b]
            pltpu.make_async_copy(
                x_hbm.at[idx_v.at[pl.ds(0, CB)]], rows, sm).wait()

        def compute(i, b):
            rows, _ = bufs[b]

            def group(g, carry2):
                e16 = g * 16 + lanes
                dl16 = dl_v[pl.ds(i * CB + g * 16, 16)]
                for t in range(16):
                    csh = (lanes + t) & 15
                    for cc in range(D // 16):
                        col = csh + cc * 16
                        vals = plsc.load_gather(rows, [e16, col])
                        plsc.addupdate_scatter(acc, [dl16, col], vals)
                return carry2

            lax.fori_loop(0, CB // 16, group, 0)

        def block(bk, carry):
            base = w * LCAP + bk * LBK
            pltpu.sync_copy(lsrc_hbm.at[pl.ds(base, LBK)], idx_v)
            pltpu.sync_copy(ldst_hbm.at[pl.ds(base, LBK)], dl_v)
            start(0, 0)

            def pair(p, carry2):
                i0 = 2 * p
                start(i0 + 1, 1)
                wait(0)
                compute(i0, 0)

                @pl.when(p < LBK // CB // 2 - 1)
                def _():
                    start(i0 + 2, 0)

                wait(1)
                compute(i0 + 1, 1)
                return carry2

            lax.fori_loop(0, LBK // CB // 2, pair, 0)
            return carry

        lax.fori_loop(0, NBLK, block, 0)
        pltpu.sync_copy(acc, agg_out.at[pl.ds(w * AR, AR)])

    return _sc_kernel(body, out_type, scratch)


@functools.lru_cache(maxsize=None)
def _make_scores():
    EPW = CPW * CB
    out_type = [jax.ShapeDtypeStruct((EP,), jnp.float32),
                jax.ShapeDtypeStruct((EP,), jnp.float32)]
    scratch = [
        pltpu.VMEM((EPW,), jnp.int32),
        pltpu.VMEM((EPW,), jnp.int32),
        pltpu.VMEM((CB, D), jnp.float32),
        pltpu.VMEM((CB, D), jnp.float32),
        pltpu.VMEM((CB, D), jnp.float32),
        pltpu.VMEM((CB, D), jnp.float32),
        pltpu.VMEM((EPW,), jnp.float32),
        pltpu.SemaphoreType.DMA,
        pltpu.SemaphoreType.DMA,
    ]

    def body(h_hbm, src_hbm, dst_hbm, nsrc_hbm, ndst_hbm, pos_out, neg_out,
             idx_a, idx_b, ra0, rb0, ra1, rb1, out_v, sem0, sem1):
        c = lax.axis_index("c")
        s = lax.axis_index("s")
        w = c * NS + s
        lanes = lax.broadcasted_iota(jnp.int32, (16,), 0)
        bufs = ((ra0, rb0, sem0), (ra1, rb1, sem1))

        def edge_set(sref, dref, oref):
            pltpu.sync_copy(sref.at[pl.ds(w * EPW, EPW)], idx_a)
            pltpu.sync_copy(dref.at[pl.ds(w * EPW, EPW)], idx_b)

            def start(i, b):
                ra, rb, sm = bufs[b]
                pltpu.async_copy(h_hbm.at[idx_a.at[pl.ds(i * CB, CB)]],
                                 ra, sm)
                pltpu.async_copy(h_hbm.at[idx_b.at[pl.ds(i * CB, CB)]],
                                 rb, sm)

            def wait(b):
                ra, rb, sm = bufs[b]
                pltpu.make_async_copy(
                    h_hbm.at[idx_a.at[pl.ds(0, CB)]], ra, sm).wait()
                pltpu.make_async_copy(
                    h_hbm.at[idx_b.at[pl.ds(0, CB)]], rb, sm).wait()

            def compute(i, b):
                ra, rb, _ = bufs[b]

                def group(g, carry2):
                    edge_ids = g * 16 + lanes
                    score = jnp.zeros((16,), jnp.float32)
                    for t in range(D):
                        col = (lanes + t) & (D - 1)
                        a = plsc.load_gather(ra, [edge_ids, col])
                        b2 = plsc.load_gather(rb, [edge_ids, col])
                        score = score + a * b2
                    out_v[pl.ds(i * CB + g * 16, 16)] = score
                    return carry2

                lax.fori_loop(0, CB // 16, group, 0)

            start(0, 0)

            def pair(p, carry):
                i0 = 2 * p
                start(i0 + 1, 1)
                wait(0)
                compute(i0, 0)

                @pl.when(p < CPW // 2 - 1)
                def _():
                    start(i0 + 2, 0)

                wait(1)
                compute(i0 + 1, 1)
                return carry

            lax.fori_loop(0, CPW // 2, pair, 0)
            pltpu.sync_copy(out_v, oref.at[pl.ds(w * EPW, EPW)])

        edge_set(src_hbm, dst_hbm, pos_out)
        edge_set(nsrc_hbm, ndst_hbm, neg_out)

    return _sc_kernel(body, out_type, scratch)


def _tc_layer(aggp, degp, x, ws_t, wn_t, b, relu):
    R = 1000

    def body(agg_ref, deg_ref, x_ref, ws_ref, wn_ref, b_ref, o_ref):
        agg = agg_ref[0] + agg_ref[1]
        deg = jnp.sum(deg_ref[0] + deg_ref[1], axis=-1, keepdims=True)
        mean = agg / jnp.maximum(deg, 1.0)
        h = (jnp.dot(x_ref[...], ws_ref[...],
                     preferred_element_type=jnp.float32) +
             jnp.dot(mean, wn_ref[...], preferred_element_type=jnp.float32) +
             b_ref[...])
        o_ref[...] = jnp.maximum(h, 0.0) if relu else h

    return pl.pallas_call(
        body,
        grid=(N // R,),
        in_specs=[
            pl.BlockSpec((NC, R, D), lambda i: (0, i, 0)),
            pl.BlockSpec((NC, R, 16), lambda i: (0, i, 0)),
            pl.BlockSpec((R, D), lambda i: (i, 0)),
            pl.BlockSpec((D, D), lambda i: (0, 0)),
            pl.BlockSpec((D, D), lambda i: (0, 0)),
            pl.BlockSpec((1, D), lambda i: (0, 0)),
        ],
        out_specs=pl.BlockSpec((R, D), lambda i: (i, 0)),
        out_shape=jax.ShapeDtypeStruct((N, D), jnp.float32),
    )(aggp, degp, x, ws_t, wn_t, b)


def _trim(part, width):
    return part.reshape(NC, NS, AR, width)[:, :, :RPT, :].reshape(
        NC, N2, width)


def kernel(feat, edge_index, neg_edge_index, W1_self, b1_self, W1_neigh,
           W2_self, b2_self, W2_neigh):
    src = edge_index[0].astype(jnp.int32)
    dst = edge_index[1].astype(jnp.int32)
    nsrc = neg_edge_index[0].astype(jnp.int32)
    ndst = neg_edge_index[1].astype(jnp.int32)

    ar = jnp.arange(PAD, dtype=jnp.int32)
    pad_gather = (ar * 131) % N
    pad_drop = jnp.full((PAD,), BIGDST, jnp.int32)
    srcp = jnp.concatenate([src, pad_gather])
    dstp = jnp.concatenate([dst, pad_drop])
    s_src = jnp.concatenate([src, pad_gather])
    s_dst = jnp.concatenate([dst, pad_gather])
    s_nsrc = jnp.concatenate([nsrc, pad_gather])
    s_ndst = jnp.concatenate([ndst, pad_gather])

    zrow = jnp.zeros((CB, D), jnp.float32)
    zdeg = jnp.zeros((CB, 16), jnp.float32)

    lsrc, ldst, degp = _make_route()(srcp, dstp, zdeg)
    degp = _trim(degp, 16)
    agg1p, = _make_agg()(feat, lsrc, ldst, zrow)
    agg1p = _trim(agg1p, D)
    h1 = _tc_layer(agg1p, degp, feat, W1_self.T, W1_neigh.T,
                   b1_self.reshape(1, D), True)
    agg2p, = _make_agg()(h1, lsrc, ldst, zrow)
    agg2p = _trim(agg2p, D)
    h2 = _tc_layer(agg2p, degp, h1, W2_self.T, W2_neigh.T,
                   b2_self.reshape(1, D), False)
    pos, neg = _make_scores()(h2, s_src, s_dst, s_nsrc, s_ndst)
    return pos[:E, None], neg[:E, None]


# speedup vs baseline: 4.0439x; 1.0642x over previous
import functools

import jax
import jax.numpy as jnp
from jax import lax
from jax.experimental import pallas as pl
from jax.experimental.pallas import tpu as pltpu
from jax.experimental.pallas import tpu_sc as plsc

N = 10000
D = 128
E = 320000
NC = 2
NS = 16
NW = NC * NS
CB = 128
CPW = 80
EP = NW * CB * CPW
PAD = EP - E
EPH = EP // NC
NCH = EPH // CB
N2 = 10112
RPT = N2 // NS
AR = RPT + 8
LCAP = 11264
LCH = LCAP // CB
BIGDST = 1 << 28
RBK = 2048
GRP = 128


@functools.lru_cache(maxsize=None)
def _mesh():
    return plsc.VectorSubcoreMesh(core_axis_name="c", subcore_axis_name="s",
                                  num_cores=NC, num_subcores=NS)


def _sc_kernel(body, out_type, scratch):
    return pl.kernel(body, out_type=out_type, mesh=_mesh(),
                     scratch_types=scratch,
                     compiler_params=pltpu.CompilerParams(
                         needs_layout_passes=False))


@functools.lru_cache(maxsize=None)
def _make_route():
    out_type = [jax.ShapeDtypeStruct((NW * LCAP,), jnp.int32),
                jax.ShapeDtypeStruct((NW * LCAP,), jnp.int32),
                jax.ShapeDtypeStruct((NW * AR, 16), jnp.float32)]
    scratch = [
        pltpu.VMEM((RBK,), jnp.int32),
        pltpu.VMEM((RBK,), jnp.int32),
        pltpu.VMEM((LCAP + 16,), jnp.int32),
        pltpu.VMEM((LCAP + 16,), jnp.int32),
        pltpu.VMEM((AR, 16), jnp.float32),
        pltpu.SemaphoreType.DMA,
    ]

    def body(src_hbm, dst_hbm, zdeg_hbm, lsrc_out, ldst_out, deg_out,
             src_v, dst_v, lsrc, ldst, dega, sem):
        c = lax.axis_index("c")
        s = lax.axis_index("s")
        w = c * NS + s
        lanes = lax.broadcasted_iota(jnp.int32, (16,), 0)
        lo = s * RPT
        hi = lo + RPT

        for q in range(AR // CB):
            pltpu.sync_copy(zdeg_hbm, dega.at[pl.ds(q * CB, CB)])

        dummy_dst = RPT + (lanes & 7)

        def prefill(k, carry):
            dummy_src = ((k * 16 + lanes) * 131) & 8191
            lsrc[pl.ds(k * 16, 16)] = dummy_src
            ldst[pl.ds(k * 16, 16)] = dummy_dst
            return carry

        lax.fori_loop(0, LCAP // 16, prefill, 0)

        def chunk(i, cnt):
            base = c * EPH + i * RBK
            pltpu.sync_copy(src_hbm.at[pl.ds(base, RBK)], src_v)
            pltpu.sync_copy(dst_hbm.at[pl.ds(base, RBK)], dst_v)

            def sub(j, cnt2):
                return process(j, cnt2)
            return lax.fori_loop(0, RBK // GRP, sub, cnt)

        def process(g, cnt):
            if True:
                for u in range(GRP // 16):
                    s16 = src_v[pl.ds(g * GRP + u * 16, 16)]
                    d16 = dst_v[pl.ds(g * GRP + u * 16, 16)]
                    inr = (d16 >= lo) & (d16 < hi)
                    dloc = jnp.where(inr, d16 - lo, 0)
                    plsc.addupdate_scatter(dega, [dloc, lanes],
                                           jnp.ones((16,), jnp.float32),
                                           mask=inr)
                    plsc.store_compressed(lsrc.at[pl.ds(cnt, 16)], s16,
                                          mask=inr)
                    plsc.store_compressed(ldst.at[pl.ds(cnt, 16)], dloc,
                                          mask=inr)
                    cnt = cnt + jnp.max(
                        plsc.all_reduce_population_count(inr))
            return cnt

        lax.fori_loop(0, EPH // RBK, chunk, jnp.int32(0))

        pltpu.sync_copy(lsrc.at[pl.ds(0, LCAP)],
                        lsrc_out.at[pl.ds(w * LCAP, LCAP)])
        pltpu.sync_copy(ldst.at[pl.ds(0, LCAP)],
                        ldst_out.at[pl.ds(w * LCAP, LCAP)])
        pltpu.sync_copy(dega, deg_out.at[pl.ds(w * AR, AR)])

    return _sc_kernel(body, out_type, scratch)


@functools.lru_cache(maxsize=None)
def _make_agg():
    LBK = 1024
    NBLK = LCAP // LBK
    out_type = [jax.ShapeDtypeStruct((NW * AR, D), jnp.float32)]
    scratch = [
        pltpu.VMEM((LBK,), jnp.int32),
        pltpu.VMEM((LBK,), jnp.int32),
        pltpu.VMEM((CB, D), jnp.float32),
        pltpu.VMEM((CB, D), jnp.float32),
        pltpu.VMEM((AR, D), jnp.float32),
        pltpu.SemaphoreType.DMA,
        pltpu.SemaphoreType.DMA,
    ]

    def body(x_hbm, lsrc_hbm, ldst_hbm, zrow_hbm,
             agg_out, idx_v, dl_v, rows0, rows1, acc, sem0, sem1):
        c = lax.axis_index("c")
        s = lax.axis_index("s")
        w = c * NS + s
        lanes = lax.broadcasted_iota(jnp.int32, (16,), 0)
        bufs = ((rows0, sem0), (rows1, sem1))

        for q in range(AR // CB):
            pltpu.sync_copy(zrow_hbm, acc.at[pl.ds(q * CB, CB)])

        def start(i, b):
            rows, sm = bufs[b]
            pltpu.async_copy(x_hbm.at[idx_v.at[pl.ds(i * CB, CB)]], rows, sm)

        def wait(b):
            rows, sm = bufs[b]
            pltpu.make_async_copy(
                x_hbm.at[idx_v.at[pl.ds(0, CB)]], rows, sm).wait()

        colc = tuple(lanes + cc * 16 for cc in range(D // 16))

        def compute(i, b):
            rows, _ = bufs[b]

            def group(g, carry2):
                for j in range(16):
                    e = g * 16 + j
                    ridx = plsc.load_gather(
                        dl_v, [jnp.full((16,), i * CB + e, jnp.int32)])
                    for cc in range(D // 16):
                        vals = rows[e, pl.ds(cc * 16, 16)]
                        plsc.addupdate_scatter(acc, [ridx, colc[cc]], vals)
                return carry2

            lax.fori_loop(0, CB // 16, group, 0)

        def block(bk, carry):
            base = w * LCAP + bk * LBK
            pltpu.sync_copy(lsrc_hbm.at[pl.ds(base, LBK)], idx_v)
            pltpu.sync_copy(ldst_hbm.at[pl.ds(base, LBK)], dl_v)
            start(0, 0)

            def pair(p, carry2):
                i0 = 2 * p
                start(i0 + 1, 1)
                wait(0)
                compute(i0, 0)

                @pl.when(p < LBK // CB // 2 - 1)
                def _():
                    start(i0 + 2, 0)

                wait(1)
                compute(i0 + 1, 1)
                return carry2

            lax.fori_loop(0, LBK // CB // 2, pair, 0)
            return carry

        lax.fori_loop(0, NBLK, block, 0)
        pltpu.sync_copy(acc, agg_out.at[pl.ds(w * AR, AR)])

    return _sc_kernel(body, out_type, scratch)


@functools.lru_cache(maxsize=None)
def _make_scores():
    EPW = CPW * CB
    out_type = [jax.ShapeDtypeStruct((EP,), jnp.float32),
                jax.ShapeDtypeStruct((EP,), jnp.float32)]
    scratch = [
        pltpu.VMEM((EPW,), jnp.int32),
        pltpu.VMEM((EPW,), jnp.int32),
        pltpu.VMEM((CB, D), jnp.float32),
        pltpu.VMEM((CB, D), jnp.float32),
        pltpu.VMEM((CB, D), jnp.float32),
        pltpu.VMEM((CB, D), jnp.float32),
        pltpu.VMEM((EPW,), jnp.float32),
        pltpu.SemaphoreType.DMA,
        pltpu.SemaphoreType.DMA,
    ]

    def body(h_hbm, src_hbm, dst_hbm, nsrc_hbm, ndst_hbm, pos_out, neg_out,
             idx_a, idx_b, ra0, rb0, ra1, rb1, out_v, sem0, sem1):
        c = lax.axis_index("c")
        s = lax.axis_index("s")
        w = c * NS + s
        lanes = lax.broadcasted_iota(jnp.int32, (16,), 0)
        bufs = ((ra0, rb0, sem0), (ra1, rb1, sem1))

        def edge_set(sref, dref, oref):
            pltpu.sync_copy(sref.at[pl.ds(w * EPW, EPW)], idx_a)
            pltpu.sync_copy(dref.at[pl.ds(w * EPW, EPW)], idx_b)

            def start(i, b):
                ra, rb, sm = bufs[b]
                pltpu.async_copy(h_hbm.at[idx_a.at[pl.ds(i * CB, CB)]],
                                 ra, sm)
                pltpu.async_copy(h_hbm.at[idx_b.at[pl.ds(i * CB, CB)]],
                                 rb, sm)

            def wait(b):
                ra, rb, sm = bufs[b]
                pltpu.make_async_copy(
                    h_hbm.at[idx_a.at[pl.ds(0, CB)]], ra, sm).wait()
                pltpu.make_async_copy(
                    h_hbm.at[idx_b.at[pl.ds(0, CB)]], rb, sm).wait()

            def compute(i, b):
                ra, rb, _ = bufs[b]

                def group(g, carry2):
                    edge_ids = g * 16 + lanes
                    score = jnp.zeros((16,), jnp.float32)
                    for t in range(D):
                        col = (lanes + t) & (D - 1)
                        a = plsc.load_gather(ra, [edge_ids, col])
                        b2 = plsc.load_gather(rb, [edge_ids, col])
                        score = score + a * b2
                    out_v[pl.ds(i * CB + g * 16, 16)] = score
                    return carry2

                lax.fori_loop(0, CB // 16, group, 0)

            start(0, 0)

            def pair(p, carry):
                i0 = 2 * p
                start(i0 + 1, 1)
                wait(0)
                compute(i0, 0)

                @pl.when(p < CPW // 2 - 1)
                def _():
                    start(i0 + 2, 0)

                wait(1)
                compute(i0 + 1, 1)
                return carry

            lax.fori_loop(0, CPW // 2, pair, 0)
            pltpu.sync_copy(out_v, oref.at[pl.ds(w * EPW, EPW)])

        edge_set(src_hbm, dst_hbm, pos_out)
        edge_set(nsrc_hbm, ndst_hbm, neg_out)

    return _sc_kernel(body, out_type, scratch)


def _tc_layer(aggp, degp, x, ws_t, wn_t, b, relu):
    R = 1000

    def body(agg_ref, deg_ref, x_ref, ws_ref, wn_ref, b_ref, o_ref):
        agg = agg_ref[0] + agg_ref[1]
        deg = jnp.sum(deg_ref[0] + deg_ref[1], axis=-1, keepdims=True)
        mean = agg / jnp.maximum(deg, 1.0)
        h = (jnp.dot(x_ref[...], ws_ref[...],
                     preferred_element_type=jnp.float32) +
             jnp.dot(mean, wn_ref[...], preferred_element_type=jnp.float32) +
             b_ref[...])
        o_ref[...] = jnp.maximum(h, 0.0) if relu else h

    return pl.pallas_call(
        body,
        grid=(N // R,),
        in_specs=[
            pl.BlockSpec((NC, R, D), lambda i: (0, i, 0)),
            pl.BlockSpec((NC, R, 16), lambda i: (0, i, 0)),
            pl.BlockSpec((R, D), lambda i: (i, 0)),
            pl.BlockSpec((D, D), lambda i: (0, 0)),
            pl.BlockSpec((D, D), lambda i: (0, 0)),
            pl.BlockSpec((1, D), lambda i: (0, 0)),
        ],
        out_specs=pl.BlockSpec((R, D), lambda i: (i, 0)),
        out_shape=jax.ShapeDtypeStruct((N, D), jnp.float32),
    )(aggp, degp, x, ws_t, wn_t, b)


def _trim(part, width):
    return part.reshape(NC, NS, AR, width)[:, :, :RPT, :].reshape(
        NC, N2, width)


def kernel(feat, edge_index, neg_edge_index, W1_self, b1_self, W1_neigh,
           W2_self, b2_self, W2_neigh):
    src = edge_index[0].astype(jnp.int32)
    dst = edge_index[1].astype(jnp.int32)
    nsrc = neg_edge_index[0].astype(jnp.int32)
    ndst = neg_edge_index[1].astype(jnp.int32)

    ar = jnp.arange(PAD, dtype=jnp.int32)
    pad_gather = (ar * 131) % N
    pad_drop = jnp.full((PAD,), BIGDST, jnp.int32)
    srcp = jnp.concatenate([src, pad_gather])
    dstp = jnp.concatenate([dst, pad_drop])
    s_src = jnp.concatenate([src, pad_gather])
    s_dst = jnp.concatenate([dst, pad_gather])
    s_nsrc = jnp.concatenate([nsrc, pad_gather])
    s_ndst = jnp.concatenate([ndst, pad_gather])

    zrow = jnp.zeros((CB, D), jnp.float32)
    zdeg = jnp.zeros((CB, 16), jnp.float32)

    lsrc, ldst, degp = _make_route()(srcp, dstp, zdeg)
    degp = _trim(degp, 16)
    agg1p, = _make_agg()(feat, lsrc, ldst, zrow)
    agg1p = _trim(agg1p, D)
    h1 = _tc_layer(agg1p, degp, feat, W1_self.T, W1_neigh.T,
                   b1_self.reshape(1, D), True)
    agg2p, = _make_agg()(h1, lsrc, ldst, zrow)
    agg2p = _trim(agg2p, D)
    h2 = _tc_layer(agg2p, degp, h1, W2_self.T, W2_neigh.T,
                   b2_self.reshape(1, D), False)
    pos, neg = _make_scores()(h2, s_src, s_dst, s_nsrc, s_ndst)
    return pos[:E, None], neg[:E, None]
